# baseline (speedup 1.0000x reference)
import functools

import jax
import jax.numpy as jnp
from jax import lax
from jax.experimental import pallas as pl
from jax.experimental.pallas import tpu as pltpu
from jax.experimental.pallas import tpu_sc as plsc

N = 100000
D = 8
PROJ = 64
POS_DIM = 3
NUM_SEGMENTS = 512
ROW = D * PROJ

WBLK = 2000

NCORE = 2
NSUB = 16
NW = NCORE * NSUB
CH = 3136
BB = 32
NB_EFF = 99
NBUF = 3
RING = 16
DUMMY = NUM_SEGMENTS
ACC_ROWS = 528
NVEC = ROW // 16
NEG = -2147483647


def _weights_body(pos_ref, w1_ref, b1_ref, w2_ref, b2_ref, out_ref):
    pos = pos_ref[...]
    h = jnp.dot(pos, w1_ref[...], preferred_element_type=jnp.float32) + b1_ref[...]
    h = jnp.where(h > 0, h, jnp.exp(h) - 1.0)
    w = jnp.dot(h, w2_ref[...], preferred_element_type=jnp.float32) + b2_ref[...]
    w = w - jnp.max(w, axis=-1, keepdims=True)
    w = jnp.exp(w)
    out_ref[...] = w / jnp.sum(w, axis=-1, keepdims=True)


def _compute_weights(positions, W1, b1, W2, b2):
    nblocks = N // WBLK
    return pl.pallas_call(
        _weights_body,
        grid=(nblocks,),
        in_specs=[
            pl.BlockSpec((WBLK, POS_DIM), lambda i: (i, 0)),
            pl.BlockSpec((POS_DIM, PROJ), lambda i: (0, 0)),
            pl.BlockSpec((1, PROJ), lambda i: (0, 0)),
            pl.BlockSpec((PROJ, PROJ), lambda i: (0, 0)),
            pl.BlockSpec((1, PROJ), lambda i: (0, 0)),
        ],
        out_specs=pl.BlockSpec((WBLK, PROJ), lambda i: (i, 0)),
        out_shape=jax.ShapeDtypeStruct((N, PROJ), jnp.float32),
    )(positions, W1, b1.reshape(1, PROJ), W2, b2.reshape(1, PROJ))


def _combine_body(p_ref, out_ref):
    out_ref[...] = p_ref[0:NUM_SEGMENTS, :] + p_ref[NUM_SEGMENTS:2 * NUM_SEGMENTS, :]


def _combine(partials):
    return pl.pallas_call(
        _combine_body,
        out_shape=jax.ShapeDtypeStruct((NUM_SEGMENTS, ROW), jnp.float32),
    )(partials)


def _sc_pool_body(x_hbm, w_hbm, ids_hbm, out_hbm,
                  xbuf, wbuf, idbuf, ring, idxv, accsh, semx, semw, semi):
    c = lax.axis_index("c")
    s = lax.axis_index("s")
    wid = c * NSUB + s
    r0 = wid * CH
    nrows = jnp.minimum(CH, N - r0)
    nb = nrows // BB

    zero16 = jnp.zeros((16,), jnp.float32)
    iota16 = lax.iota(jnp.int32, 16)

    def _zring(r, _):
        for j in range(NVEC):
            ring[r, pl.ds(16 * j, 16)] = zero16
        return 0
    lax.fori_loop(0, RING, _zring, 0)
    z0 = s * 32
    pltpu.sync_copy(ring, accsh.at[pl.ds(z0, 16)])
    pltpu.sync_copy(ring, accsh.at[pl.ds(z0 + 16, 16)])

    @pl.when(s == 0)
    def _zdummy():
        pltpu.sync_copy(ring, accsh.at[pl.ds(NUM_SEGMENTS, 16)])
    plsc.subcore_barrier()

    def _issue(b, p):
        bc = jnp.minimum(b, nb - 1)
        base = r0 + bc * BB
        pltpu.make_async_copy(x_hbm.at[pl.ds(base, BB), :], xbuf.at[p], semx.at[p]).start()
        pltpu.make_async_copy(w_hbm.at[pl.ds(base, BB), :], wbuf.at[p], semw.at[p]).start()
        pltpu.make_async_copy(ids_hbm.at[pl.ds(base, BB)], idbuf.at[p], semi.at[p]).start()

    def _wait(b, p):
        bc = jnp.minimum(b, nb - 1)
        base = r0 + bc * BB
        pltpu.make_async_copy(x_hbm.at[pl.ds(base, BB), :], xbuf.at[p], semx.at[p]).wait()
        pltpu.make_async_copy(w_hbm.at[pl.ds(base, BB), :], wbuf.at[p], semw.at[p]).wait()
        pltpu.make_async_copy(ids_hbm.at[pl.ds(base, BB)], idbuf.at[p], semi.at[p]).wait()

    def _accum_row(acc, p, r):
        wv = [wbuf[p, r, pl.ds(16 * g, 16)] for g in range(4)]
        return tuple(acc[j] + xbuf[p, r, pl.ds(16 * j, 16)] * wv[j % 4]
                     for j in range(NVEC))

    def _drain(k, ids_vec):
        idxv[...] = ids_vec
        pltpu.sync_copy(ring, accsh.at[idxv], add=True)
        return jnp.int32(0), jnp.full((16,), DUMMY, jnp.int32)

    def _flush(cur_seg, k, ids_vec, acc):
        idx = jnp.where(cur_seg < 0, jnp.int32(DUMMY), cur_seg)

        def _do_drain(args):
            return _drain(*args)
        k, ids_vec = lax.cond(k == RING, _do_drain, lambda a: a, (k, ids_vec))
        for j in range(NVEC):
            ring[k, pl.ds(16 * j, 16)] = acc[j]
        ids_vec = jnp.where(iota16 == k, idx, ids_vec)
        return k + 1, ids_vec, tuple(zero16 for _ in range(NVEC))

    def _group(carry, p, b, g):
        cur_seg, k, ids_vec = carry[0], carry[1], carry[2]
        acc = carry[3:]
        gids = idbuf[p, pl.ds(16 * g, 16)]
        gids = jnp.where(b < nb, gids, jnp.full((16,), DUMMY, jnp.int32))
        first = jnp.min(gids)
        last = jnp.max(gids)

        def _fast(op):
            cur_seg, k, ids_vec = op[0], op[1], op[2]
            acc = op[3:]

            def _new_run(args):
                cur_seg, k, ids_vec = args[0], args[1], args[2]
                k, ids_vec, acc = _flush(cur_seg, k, ids_vec, args[3:])
                return (first, k, ids_vec) + acc
            op = lax.cond(first != cur_seg, _new_run, lambda a: a,
                          (cur_seg, k, ids_vec) + acc)
            cur_seg, k, ids_vec = op[0], op[1], op[2]
            acc = op[3:]
            for i in range(16):
                acc = _accum_row(acc, p, 16 * g + i)
            return (cur_seg, k, ids_vec) + acc

        def _slow(op):
            def _row(i, op):
                cur_seg, k, ids_vec = op[0], op[1], op[2]
                sid = jnp.max(jnp.where(iota16 == i, gids, jnp.int32(NEG)))

                def _new_run(args):
                    cur_seg, k, ids_vec = args[0], args[1], args[2]
                    k, ids_vec, acc = _flush(cur_seg, k, ids_vec, args[3:])
                    return (sid, k, ids_vec) + acc
                op = lax.cond(sid != cur_seg, _new_run, lambda a: a, op)
                acc = _accum_row(op[3:], p, 16 * g + i)
                return op[:3] + acc
            return lax.fori_loop(0, 16, _row, op)

        return lax.cond(first == last, _fast, _slow, (cur_seg, k, ids_vec) + acc)

    for p in range(NBUF):
        _issue(jnp.int32(p), p)

    init = (jnp.int32(-1), jnp.int32(0), jnp.full((16,), DUMMY, jnp.int32)) + \
        tuple(zero16 for _ in range(NVEC))

    def _super(i, carry):
        for p in range(NBUF):
            b = NBUF * i + p
            _wait(b, p)
            carry = _group(carry, p, b, 0)
            carry = _group(carry, p, b, 1)

            @pl.when(b + NBUF < NB_EFF)
            def _():
                _issue(b + NBUF, p)
        return carry

    carry = lax.fori_loop(0, NB_EFF // NBUF, _super, init)

    cur_seg, k, ids_vec = carry[0], carry[1], carry[2]
    k, ids_vec, _ = _flush(cur_seg, k, ids_vec, carry[3:])
    _drain(k, ids_vec)

    plsc.subcore_barrier()

    o0 = c * NUM_SEGMENTS + s * 32
    pltpu.sync_copy(accsh.at[pl.ds(s * 32, 32)], out_hbm.at[pl.ds(o0, 32), :])


@functools.partial(
    pl.kernel,
    out_type=jax.ShapeDtypeStruct((NCORE * NUM_SEGMENTS, ROW), jnp.float32),
    mesh=plsc.VectorSubcoreMesh(core_axis_name="c", subcore_axis_name="s"),
    scratch_types=[
        pltpu.VMEM((NBUF, BB, ROW), jnp.float32),
        pltpu.VMEM((NBUF, BB, PROJ), jnp.float32),
        pltpu.VMEM((NBUF, BB), jnp.int32),
        pltpu.VMEM((RING, ROW), jnp.float32),
        pltpu.VMEM((16,), jnp.int32),
        pltpu.VMEM_SHARED((ACC_ROWS, ROW), jnp.float32),
        pltpu.SemaphoreType.DMA((NBUF,)),
        pltpu.SemaphoreType.DMA((NBUF,)),
        pltpu.SemaphoreType.DMA((NBUF,)),
    ],
    compiler_params=pltpu.CompilerParams(
        use_tc_tiling_on_sc=False, needs_layout_passes=False),
)
def _sc_pool(x_hbm, w_hbm, ids_hbm, out_hbm, *scratch):
    _sc_pool_body(x_hbm, w_hbm, ids_hbm, out_hbm, *scratch)


def kernel(x, batch, positions, W1, b1, W2, b2):
    weights = _compute_weights(positions, W1, b1, W2, b2)
    ids = batch.astype(jnp.int32)
    partials = _sc_pool(x.reshape(N, ROW), weights, ids)
    pooled = _combine(partials)
    out = pooled.reshape(NUM_SEGMENTS, D, PROJ)
    return jnp.transpose(out, (0, 2, 1))[..., None]


# speedup vs baseline: 1.2162x; 1.2162x over previous
import functools

import jax
import jax.numpy as jnp
from jax import lax
from jax.experimental import pallas as pl
from jax.experimental.pallas import tpu as pltpu
from jax.experimental.pallas import tpu_sc as plsc

N = 100000
D = 8
PROJ = 64
POS_DIM = 3
NUM_SEGMENTS = 512
ROW = D * PROJ

WBLK = 2000

NCORE = 2
NSUB = 16
NW = NCORE * NSUB
CH = 3136
BB = 32
NB_EFF = 98
NBUF = 2
RING = 16
DUMMY = NUM_SEGMENTS
ACC_ROWS = 528
NVEC = ROW // 16
NEG = -2147483647


def _weights_body(pos_ref, w1_ref, b1_ref, w2_ref, b2_ref, out_ref):
    pos = pos_ref[...]
    h = jnp.dot(pos, w1_ref[...], preferred_element_type=jnp.float32) + b1_ref[...]
    h = jnp.where(h > 0, h, jnp.exp(h) - 1.0)
    w = jnp.dot(h, w2_ref[...], preferred_element_type=jnp.float32) + b2_ref[...]
    w = w - jnp.max(w, axis=-1, keepdims=True)
    w = jnp.exp(w)
    out_ref[...] = w / jnp.sum(w, axis=-1, keepdims=True)


def _compute_weights(positions, W1, b1, W2, b2):
    nblocks = N // WBLK
    return pl.pallas_call(
        _weights_body,
        grid=(nblocks,),
        in_specs=[
            pl.BlockSpec((WBLK, POS_DIM), lambda i: (i, 0)),
            pl.BlockSpec((POS_DIM, PROJ), lambda i: (0, 0)),
            pl.BlockSpec((1, PROJ), lambda i: (0, 0)),
            pl.BlockSpec((PROJ, PROJ), lambda i: (0, 0)),
            pl.BlockSpec((1, PROJ), lambda i: (0, 0)),
        ],
        out_specs=pl.BlockSpec((WBLK, PROJ), lambda i: (i, 0)),
        out_shape=jax.ShapeDtypeStruct((N, PROJ), jnp.float32),
    )(positions, W1, b1.reshape(1, PROJ), W2, b2.reshape(1, PROJ))


def _combine_body(p_ref, out_ref):
    out_ref[...] = p_ref[0:NUM_SEGMENTS, :] + p_ref[NUM_SEGMENTS:2 * NUM_SEGMENTS, :]


def _combine(partials):
    return pl.pallas_call(
        _combine_body,
        out_shape=jax.ShapeDtypeStruct((NUM_SEGMENTS, ROW), jnp.float32),
    )(partials)


def _sc_pool_body(x_hbm, w_hbm, ids_hbm, out_hbm,
                  xbuf, wbuf, idbuf, ring, idxv, accbuf, accsh,
                  semx, semw, semi):
    c = lax.axis_index("c")
    s = lax.axis_index("s")
    wid = c * NSUB + s
    r0 = wid * CH
    nrows = jnp.minimum(CH, N - r0)
    nb = nrows // BB

    zero16 = jnp.zeros((16,), jnp.float32)
    iota16 = lax.iota(jnp.int32, 16)

    def _zring(r, _):
        for j in range(NVEC):
            ring[r, pl.ds(16 * j, 16)] = zero16
        return 0
    lax.fori_loop(0, RING, _zring, 0)
    z0 = s * 32
    pltpu.sync_copy(ring, accsh.at[pl.ds(z0, 16)])
    pltpu.sync_copy(ring, accsh.at[pl.ds(z0 + 16, 16)])

    @pl.when(s == 0)
    def _zdummy():
        pltpu.sync_copy(ring, accsh.at[pl.ds(NUM_SEGMENTS, 16)])
    plsc.subcore_barrier()

    def _issue(b, p):
        bc = jnp.minimum(b, nb - 1)
        base = r0 + bc * BB
        pltpu.make_async_copy(x_hbm.at[pl.ds(base, BB), :], xbuf.at[p], semx.at[p]).start()
        pltpu.make_async_copy(w_hbm.at[pl.ds(base, BB), :], wbuf.at[p], semw.at[p]).start()
        pltpu.make_async_copy(ids_hbm.at[pl.ds(base, BB)], idbuf.at[p], semi.at[p]).start()

    def _wait(b, p):
        bc = jnp.minimum(b, nb - 1)
        base = r0 + bc * BB
        pltpu.make_async_copy(x_hbm.at[pl.ds(base, BB), :], xbuf.at[p], semx.at[p]).wait()
        pltpu.make_async_copy(w_hbm.at[pl.ds(base, BB), :], wbuf.at[p], semw.at[p]).wait()
        pltpu.make_async_copy(ids_hbm.at[pl.ds(base, BB)], idbuf.at[p], semi.at[p]).wait()

    def _zero_acc():
        for j in range(NVEC):
            accbuf[pl.ds(16 * j, 16)] = zero16
    _zero_acc()

    def _drain(k, ids_vec):
        idxv[...] = ids_vec
        pltpu.sync_copy(ring, accsh.at[idxv], add=True)
        return jnp.int32(0), jnp.full((16,), DUMMY, jnp.int32)

    def _flush(cur_seg, k, ids_vec):
        idx = jnp.where(cur_seg < 0, jnp.int32(DUMMY), cur_seg)
        k, ids_vec = lax.cond(k == RING, lambda a: _drain(*a), lambda a: a,
                              (k, ids_vec))
        for j in range(NVEC):
            ring[k, pl.ds(16 * j, 16)] = accbuf[pl.ds(16 * j, 16)]
        _zero_acc()
        ids_vec = jnp.where(iota16 == k, idx, ids_vec)
        return k + 1, ids_vec

    def _accum_run16(p, g):
        acc = [zero16] * NVEC
        for i in range(16):
            r = 16 * g + i
            wv = [wbuf[p, r, pl.ds(16 * q, 16)] for q in range(4)]
            for j in range(NVEC):
                acc[j] = acc[j] + xbuf[p, r, pl.ds(16 * j, 16)] * wv[j % 4]
        for j in range(NVEC):
            plsc.addupdate(accbuf.at[pl.ds(16 * j, 16)], acc[j])

    def _accum_row(p, r):
        wv = [wbuf[p, r, pl.ds(16 * q, 16)] for q in range(4)]
        for j in range(NVEC):
            plsc.addupdate(accbuf.at[pl.ds(16 * j, 16)],
                           xbuf[p, r, pl.ds(16 * j, 16)] * wv[j % 4])

    def _group(carry, p, b, g):
        gids = idbuf[p, pl.ds(16 * g, 16)]
        gids = jnp.where(b < nb, gids, jnp.full((16,), DUMMY, jnp.int32))
        first = jnp.min(gids)
        last = jnp.max(gids)

        def _uni(op):
            cur_seg, k, ids_vec = op
            k, ids_vec = lax.cond(first != cur_seg,
                                  lambda a: _flush(*a), lambda a: a[1:],
                                  (cur_seg, k, ids_vec))
            _accum_run16(p, g)
            return first, k, ids_vec

        def _slow(op):
            def _row(i, op):
                cur_seg, k, ids_vec = op
                sid = jnp.max(jnp.where(iota16 == i, gids, jnp.int32(NEG)))
                k, ids_vec = lax.cond(sid != cur_seg,
                                      lambda a: _flush(*a), lambda a: a[1:],
                                      (cur_seg, k, ids_vec))
                _accum_row(p, 16 * g + i)
                return sid, k, ids_vec
            return lax.fori_loop(0, 16, _row, op)

        return lax.cond(first == last, _uni, _slow, carry)

    for p in range(NBUF):
        _issue(jnp.int32(p), p)

    init = (jnp.int32(-1), jnp.int32(0), jnp.full((16,), DUMMY, jnp.int32))

    def _super(i, carry):
        for p in range(NBUF):
            b = NBUF * i + p
            _wait(b, p)
            carry = lax.fori_loop(0, BB // 16,
                                  lambda g, cc: _group(cc, p, b, g), carry)

            @pl.when(b + NBUF < NB_EFF)
            def _():
                _issue(b + NBUF, p)
        return carry

    cur_seg, k, ids_vec = lax.fori_loop(0, NB_EFF // NBUF, _super, init)

    k, ids_vec = _flush(cur_seg, k, ids_vec)
    _drain(k, ids_vec)

    plsc.subcore_barrier()

    o0 = c * NUM_SEGMENTS + s * 32
    pltpu.sync_copy(accsh.at[pl.ds(s * 32, 32)], out_hbm.at[pl.ds(o0, 32), :])


@functools.partial(
    pl.kernel,
    out_type=jax.ShapeDtypeStruct((NCORE * NUM_SEGMENTS, ROW), jnp.float32),
    mesh=plsc.VectorSubcoreMesh(core_axis_name="c", subcore_axis_name="s"),
    scratch_types=[
        pltpu.VMEM((NBUF, BB, ROW), jnp.float32),
        pltpu.VMEM((NBUF, BB, PROJ), jnp.float32),
        pltpu.VMEM((NBUF, BB), jnp.int32),
        pltpu.VMEM((RING, ROW), jnp.float32),
        pltpu.VMEM((16,), jnp.int32),
        pltpu.VMEM((ROW,), jnp.float32),
        pltpu.VMEM_SHARED((ACC_ROWS, ROW), jnp.float32),
        pltpu.SemaphoreType.DMA((NBUF,)),
        pltpu.SemaphoreType.DMA((NBUF,)),
        pltpu.SemaphoreType.DMA((NBUF,)),
    ],
    compiler_params=pltpu.CompilerParams(
        use_tc_tiling_on_sc=False, needs_layout_passes=False),
)
def _sc_pool(x_hbm, w_hbm, ids_hbm, out_hbm, *scratch):
    _sc_pool_body(x_hbm, w_hbm, ids_hbm, out_hbm, *scratch)


def kernel(x, batch, positions, W1, b1, W2, b2):
    weights = _compute_weights(positions, W1, b1, W2, b2)
    ids = batch.astype(jnp.int32)
    partials = _sc_pool(x.reshape(N, ROW), weights, ids)
    pooled = _combine(partials)
    out = pooled.reshape(NUM_SEGMENTS, D, PROJ)
    return jnp.transpose(out, (0, 2, 1))[..., None]


# speedup vs baseline: 1.2383x; 1.0182x over previous
import functools

import jax
import jax.numpy as jnp
from jax import lax
from jax.experimental import pallas as pl
from jax.experimental.pallas import tpu as pltpu
from jax.experimental.pallas import tpu_sc as plsc

N = 100000
D = 8
PROJ = 64
POS_DIM = 3
NUM_SEGMENTS = 512
ROW = D * PROJ

WBLK = 2000

NCORE = 2
NSUB = 16
NW = NCORE * NSUB
CH = 3136
BB = 32
NB_EFF = 98
NBUF = 2
RING = 16
DUMMY = NUM_SEGMENTS
ACC_ROWS = 528
NVEC = ROW // 16
NEG = -2147483647


def _weights_body(pos_ref, w1_ref, b1_ref, w2_ref, b2_ref, out_ref):
    pos = pos_ref[...]
    h = jnp.dot(pos, w1_ref[...], preferred_element_type=jnp.float32) + b1_ref[...]
    h = jnp.where(h > 0, h, jnp.exp(h) - 1.0)
    w = jnp.dot(h, w2_ref[...], preferred_element_type=jnp.float32) + b2_ref[...]
    w = w - jnp.max(w, axis=-1, keepdims=True)
    w = jnp.exp(w)
    out_ref[...] = w / jnp.sum(w, axis=-1, keepdims=True)


def _compute_weights(positions, W1, b1, W2, b2):
    nblocks = N // WBLK
    return pl.pallas_call(
        _weights_body,
        grid=(nblocks,),
        in_specs=[
            pl.BlockSpec((WBLK, POS_DIM), lambda i: (i, 0)),
            pl.BlockSpec((POS_DIM, PROJ), lambda i: (0, 0)),
            pl.BlockSpec((1, PROJ), lambda i: (0, 0)),
            pl.BlockSpec((PROJ, PROJ), lambda i: (0, 0)),
            pl.BlockSpec((1, PROJ), lambda i: (0, 0)),
        ],
        out_specs=pl.BlockSpec((WBLK, PROJ), lambda i: (i, 0)),
        out_shape=jax.ShapeDtypeStruct((N, PROJ), jnp.float32),
    )(positions, W1, b1.reshape(1, PROJ), W2, b2.reshape(1, PROJ))


def _combine_body(p_ref, out_ref):
    out_ref[...] = p_ref[0:NUM_SEGMENTS, :] + p_ref[NUM_SEGMENTS:2 * NUM_SEGMENTS, :]


def _combine(partials):
    return pl.pallas_call(
        _combine_body,
        out_shape=jax.ShapeDtypeStruct((NUM_SEGMENTS, ROW), jnp.float32),
    )(partials)


def _sc_pool_body(x_hbm, w_hbm, ids_hbm, out_hbm,
                  xbuf, wbuf, idbuf, ring, idxv, accbuf, accsh,
                  semx, semw):
    c = lax.axis_index("c")
    s = lax.axis_index("s")
    wid = c * NSUB + s
    r0 = wid * CH
    nrows = jnp.minimum(CH, N - r0)
    nb = nrows // BB
    i0 = jnp.minimum(r0, N - CH)
    d0 = r0 - i0

    zero16 = jnp.zeros((16,), jnp.float32)
    iota16 = lax.iota(jnp.int32, 16)

    def _zring(r, _):
        for j in range(NVEC):
            ring[r, pl.ds(16 * j, 16)] = zero16
        return 0
    lax.fori_loop(0, RING, _zring, 0)
    z0 = s * 32
    pltpu.sync_copy(ring, accsh.at[pl.ds(z0, 16)])
    pltpu.sync_copy(ring, accsh.at[pl.ds(z0 + 16, 16)])

    @pl.when(s == 0)
    def _zdummy():
        pltpu.sync_copy(ring, accsh.at[pl.ds(NUM_SEGMENTS, 16)])
    plsc.subcore_barrier()

    pltpu.sync_copy(ids_hbm.at[pl.ds(i0, CH)], idbuf)

    def _issue(b, p):
        bc = jnp.minimum(b, nb - 1)
        base = r0 + bc * BB
        pltpu.make_async_copy(x_hbm.at[pl.ds(base, BB), :], xbuf.at[p], semx.at[p]).start()
        pltpu.make_async_copy(w_hbm.at[pl.ds(base, BB), :], wbuf.at[p], semw.at[p]).start()

    def _wait(b, p):
        bc = jnp.minimum(b, nb - 1)
        base = r0 + bc * BB
        pltpu.make_async_copy(x_hbm.at[pl.ds(base, BB), :], xbuf.at[p], semx.at[p]).wait()
        pltpu.make_async_copy(w_hbm.at[pl.ds(base, BB), :], wbuf.at[p], semw.at[p]).wait()

    def _zero_acc():
        for j in range(NVEC):
            accbuf[pl.ds(16 * j, 16)] = zero16
    _zero_acc()

    def _drain(k, ids_vec):
        idxv[...] = ids_vec
        pltpu.sync_copy(ring, accsh.at[idxv], add=True)
        return jnp.int32(0), jnp.full((16,), DUMMY, jnp.int32)

    def _flush(cur_seg, k, ids_vec):
        idx = jnp.where(cur_seg < 0, jnp.int32(DUMMY), cur_seg)
        k, ids_vec = lax.cond(k == RING, lambda a: _drain(*a), lambda a: a,
                              (k, ids_vec))
        for j in range(NVEC):
            ring[k, pl.ds(16 * j, 16)] = accbuf[pl.ds(16 * j, 16)]
        _zero_acc()
        ids_vec = jnp.where(iota16 == k, idx, ids_vec)
        return k + 1, ids_vec

    def _accum_run16(p, g):
        acc = [zero16] * NVEC
        for i in range(16):
            r = 16 * g + i
            wv = [wbuf[p, r, pl.ds(16 * q, 16)] for q in range(4)]
            for j in range(NVEC):
                acc[j] = acc[j] + xbuf[p, r, pl.ds(16 * j, 16)] * wv[j % 4]
        for j in range(NVEC):
            plsc.addupdate(accbuf.at[pl.ds(16 * j, 16)], acc[j])

    def _accum_row(p, r):
        wv = [wbuf[p, r, pl.ds(16 * q, 16)] for q in range(4)]
        for j in range(NVEC):
            plsc.addupdate(accbuf.at[pl.ds(16 * j, 16)],
                           xbuf[p, r, pl.ds(16 * j, 16)] * wv[j % 4])

    def _group(carry, p, b, g):
        pos = d0 + jnp.minimum(b * BB + 16 * g, nrows - 16)
        gids = idbuf[pl.ds(pos, 16)]
        gids = jnp.where(b < nb, gids, jnp.full((16,), DUMMY, jnp.int32))
        first = jnp.min(gids)
        last = jnp.max(gids)

        def _uni(op):
            cur_seg, k, ids_vec = op
            k, ids_vec = lax.cond(first != cur_seg,
                                  lambda a: _flush(*a), lambda a: a[1:],
                                  (cur_seg, k, ids_vec))
            _accum_run16(p, g)
            return first, k, ids_vec

        def _slow(op):
            def _row(i, op):
                cur_seg, k, ids_vec = op
                sid = jnp.max(jnp.where(iota16 == i, gids, jnp.int32(NEG)))
                k, ids_vec = lax.cond(sid != cur_seg,
                                      lambda a: _flush(*a), lambda a: a[1:],
                                      (cur_seg, k, ids_vec))
                _accum_row(p, 16 * g + i)
                return sid, k, ids_vec
            return lax.fori_loop(0, 16, _row, op)

        return lax.cond(first == last, _uni, _slow, carry)

    for p in range(NBUF):
        _issue(jnp.int32(p), p)

    init = (jnp.int32(-1), jnp.int32(0), jnp.full((16,), DUMMY, jnp.int32))

    def _super(i, carry):
        for p in range(NBUF):
            b = NBUF * i + p
            _wait(b, p)
            carry = lax.fori_loop(0, BB // 16,
                                  lambda g, cc: _group(cc, p, b, g), carry)

            @pl.when(b + NBUF < NB_EFF)
            def _():
                _issue(b + NBUF, p)
        return carry

    cur_seg, k, ids_vec = lax.fori_loop(0, NB_EFF // NBUF, _super, init)

    k, ids_vec = _flush(cur_seg, k, ids_vec)
    _drain(k, ids_vec)

    plsc.subcore_barrier()

    o0 = c * NUM_SEGMENTS + s * 32
    pltpu.sync_copy(accsh.at[pl.ds(s * 32, 32)], out_hbm.at[pl.ds(o0, 32), :])


@functools.partial(
    pl.kernel,
    out_type=jax.ShapeDtypeStruct((NCORE * NUM_SEGMENTS, ROW), jnp.float32),
    mesh=plsc.VectorSubcoreMesh(core_axis_name="c", subcore_axis_name="s"),
    scratch_types=[
        pltpu.VMEM((NBUF, BB, ROW), jnp.float32),
        pltpu.VMEM((NBUF, BB, PROJ), jnp.float32),
        pltpu.VMEM((CH,), jnp.int32),
        pltpu.VMEM((RING, ROW), jnp.float32),
        pltpu.VMEM((16,), jnp.int32),
        pltpu.VMEM((ROW,), jnp.float32),
        pltpu.VMEM_SHARED((ACC_ROWS, ROW), jnp.float32),
        pltpu.SemaphoreType.DMA((NBUF,)),
        pltpu.SemaphoreType.DMA((NBUF,)),
    ],
    compiler_params=pltpu.CompilerParams(
        use_tc_tiling_on_sc=False, needs_layout_passes=False),
)
def _sc_pool(x_hbm, w_hbm, ids_hbm, out_hbm, *scratch):
    _sc_pool_body(x_hbm, w_hbm, ids_hbm, out_hbm, *scratch)


def kernel(x, batch, positions, W1, b1, W2, b2):
    weights = _compute_weights(positions, W1, b1, W2, b2)
    ids = batch.astype(jnp.int32)
    partials = _sc_pool(x.reshape(N, ROW), weights, ids)
    pooled = _combine(partials)
    out = pooled.reshape(NUM_SEGMENTS, D, PROJ)
    return jnp.transpose(out, (0, 2, 1))[..., None]


# speedup vs baseline: 1.7724x; 1.4313x over previous
import functools

import jax
import jax.numpy as jnp
from jax import lax
from jax.experimental import pallas as pl
from jax.experimental.pallas import tpu as pltpu
from jax.experimental.pallas import tpu_sc as plsc

N = 100000
D = 8
PROJ = 64
POS_DIM = 3
NUM_SEGMENTS = 512
ROW = D * PROJ

WBLK = 2000

NCORE = 2
NSUB = 16
NW = NCORE * NSUB
CH = 3136
BB = 32
NB_EFF = 98
NBUF = 2
RING = 16
DUMMY = NUM_SEGMENTS
ACC_ROWS = 528
NVEC = ROW // 16
NEG = -2147483647


def _weights_body(pos_ref, w1_ref, b1_ref, w2_ref, b2_ref, out_ref):
    pos = pos_ref[...]
    h = jnp.dot(pos, w1_ref[...], preferred_element_type=jnp.float32) + b1_ref[...]
    h = jnp.where(h > 0, h, jnp.exp(h) - 1.0)
    w = jnp.dot(h, w2_ref[...], preferred_element_type=jnp.float32) + b2_ref[...]
    w = w - jnp.max(w, axis=-1, keepdims=True)
    w = jnp.exp(w)
    out_ref[...] = w / jnp.sum(w, axis=-1, keepdims=True)


def _compute_weights(positions, W1, b1, W2, b2):
    nblocks = N // WBLK
    return pl.pallas_call(
        _weights_body,
        grid=(nblocks,),
        in_specs=[
            pl.BlockSpec((WBLK, POS_DIM), lambda i: (i, 0)),
            pl.BlockSpec((POS_DIM, PROJ), lambda i: (0, 0)),
            pl.BlockSpec((1, PROJ), lambda i: (0, 0)),
            pl.BlockSpec((PROJ, PROJ), lambda i: (0, 0)),
            pl.BlockSpec((1, PROJ), lambda i: (0, 0)),
        ],
        out_specs=pl.BlockSpec((WBLK, PROJ), lambda i: (i, 0)),
        out_shape=jax.ShapeDtypeStruct((N, PROJ), jnp.float32),
    )(positions, W1, b1.reshape(1, PROJ), W2, b2.reshape(1, PROJ))


def _combine_body(p_ref, out_ref):
    out_ref[...] = p_ref[0:NUM_SEGMENTS, :] + p_ref[NUM_SEGMENTS:2 * NUM_SEGMENTS, :]


def _combine(partials):
    return pl.pallas_call(
        _combine_body,
        out_shape=jax.ShapeDtypeStruct((NUM_SEGMENTS, ROW), jnp.float32),
    )(partials)


def _sc_pool_body(x_hbm, w_hbm, ids_hbm, out_hbm,
                  xbuf, wbuf, idbuf, ring, idxv, accbuf, accsh,
                  semx, semw):
    c = lax.axis_index("c")
    s = lax.axis_index("s")
    wid = c * NSUB + s
    r0 = wid * CH
    nrows = jnp.minimum(CH, N - r0)
    nb = nrows // BB
    i0 = jnp.minimum(r0, N - CH)
    d0 = r0 - i0

    zero16 = jnp.zeros((16,), jnp.float32)
    iota16 = lax.iota(jnp.int32, 16)

    def _zring(r, _):
        for j in range(NVEC):
            ring[r, pl.ds(16 * j, 16)] = zero16
        return 0
    lax.fori_loop(0, RING, _zring, 0)
    z0 = s * 32
    pltpu.sync_copy(ring, accsh.at[pl.ds(z0, 16)])
    pltpu.sync_copy(ring, accsh.at[pl.ds(z0 + 16, 16)])

    @pl.when(s == 0)
    def _zdummy():
        pltpu.sync_copy(ring, accsh.at[pl.ds(NUM_SEGMENTS, 16)])
    plsc.subcore_barrier()

    pltpu.sync_copy(ids_hbm.at[pl.ds(i0, CH)], idbuf)

    def _issue(b, p):
        bc = jnp.minimum(b, nb - 1)
        base = r0 + bc * BB
        pltpu.make_async_copy(x_hbm.at[pl.ds(base, BB), :], xbuf.at[p], semx.at[p]).start()
        pltpu.make_async_copy(w_hbm.at[pl.ds(base, BB), :], wbuf.at[p], semw.at[p]).start()

    def _wait(b, p):
        bc = jnp.minimum(b, nb - 1)
        base = r0 + bc * BB
        pltpu.make_async_copy(x_hbm.at[pl.ds(base, BB), :], xbuf.at[p], semx.at[p]).wait()
        pltpu.make_async_copy(w_hbm.at[pl.ds(base, BB), :], wbuf.at[p], semw.at[p]).wait()

    def _zero_acc():
        for j in range(NVEC):
            accbuf[pl.ds(16 * j, 16)] = zero16
    _zero_acc()

    def _drain(k, ids_vec):
        idxv[...] = ids_vec
        pltpu.sync_copy(ring, accsh.at[idxv], add=True)
        return jnp.int32(0), jnp.full((16,), DUMMY, jnp.int32)

    def _flush(cur_seg, k, ids_vec):
        idx = jnp.where(cur_seg < 0, jnp.int32(DUMMY), cur_seg)
        k, ids_vec = lax.cond(k == RING, lambda a: _drain(*a), lambda a: a,
                              (k, ids_vec))
        for j in range(NVEC):
            ring[k, pl.ds(16 * j, 16)] = accbuf[pl.ds(16 * j, 16)]
        _zero_acc()
        ids_vec = jnp.where(iota16 == k, idx, ids_vec)
        return k + 1, ids_vec

    def _accum_run16(p, g):
        for q in range(4):
            acc = [zero16] * D
            for i in range(16):
                r = 16 * g + i
                wq = wbuf[p, r, pl.ds(16 * q, 16)]
                for dd in range(D):
                    acc[dd] = acc[dd] + xbuf[p, r, pl.ds(16 * (4 * dd + q), 16)] * wq
            for dd in range(D):
                plsc.addupdate(accbuf.at[pl.ds(16 * (4 * dd + q), 16)], acc[dd])

    def _accum_row(p, r):
        wv = [wbuf[p, r, pl.ds(16 * q, 16)] for q in range(4)]
        for j in range(NVEC):
            plsc.addupdate(accbuf.at[pl.ds(16 * j, 16)],
                           xbuf[p, r, pl.ds(16 * j, 16)] * wv[j % 4])

    def _group(carry, p, b, g):
        pos = d0 + jnp.minimum(b * BB + 16 * g, nrows - 16)
        gids = idbuf[pl.ds(pos, 16)]
        gids = jnp.where(b < nb, gids, jnp.full((16,), DUMMY, jnp.int32))
        first = jnp.min(gids)
        last = jnp.max(gids)

        def _uni(op):
            cur_seg, k, ids_vec = op
            k, ids_vec = lax.cond(first != cur_seg,
                                  lambda a: _flush(*a), lambda a: a[1:],
                                  (cur_seg, k, ids_vec))
            _accum_run16(p, g)
            return first, k, ids_vec

        def _slow(op):
            def _row(i, op):
                cur_seg, k, ids_vec = op
                sid = jnp.max(jnp.where(iota16 == i, gids, jnp.int32(NEG)))
                k, ids_vec = lax.cond(sid != cur_seg,
                                      lambda a: _flush(*a), lambda a: a[1:],
                                      (cur_seg, k, ids_vec))
                _accum_row(p, 16 * g + i)
                return sid, k, ids_vec
            return lax.fori_loop(0, 16, _row, op)

        return lax.cond(first == last, _uni, _slow, carry)

    for p in range(NBUF):
        _issue(jnp.int32(p), p)

    init = (jnp.int32(-1), jnp.int32(0), jnp.full((16,), DUMMY, jnp.int32))

    def _super(i, carry):
        for p in range(NBUF):
            b = NBUF * i + p
            _wait(b, p)
            carry = lax.fori_loop(0, BB // 16,
                                  lambda g, cc: _group(cc, p, b, g), carry)

            @pl.when(b + NBUF < NB_EFF)
            def _():
                _issue(b + NBUF, p)
        return carry

    cur_seg, k, ids_vec = lax.fori_loop(0, NB_EFF // NBUF, _super, init)

    k, ids_vec = _flush(cur_seg, k, ids_vec)
    _drain(k, ids_vec)

    plsc.subcore_barrier()

    o0 = c * NUM_SEGMENTS + s * 32
    pltpu.sync_copy(accsh.at[pl.ds(s * 32, 32)], out_hbm.at[pl.ds(o0, 32), :])


@functools.partial(
    pl.kernel,
    out_type=jax.ShapeDtypeStruct((NCORE * NUM_SEGMENTS, ROW), jnp.float32),
    mesh=plsc.VectorSubcoreMesh(core_axis_name="c", subcore_axis_name="s"),
    scratch_types=[
        pltpu.VMEM((NBUF, BB, ROW), jnp.float32),
        pltpu.VMEM((NBUF, BB, PROJ), jnp.float32),
        pltpu.VMEM((CH,), jnp.int32),
        pltpu.VMEM((RING, ROW), jnp.float32),
        pltpu.VMEM((16,), jnp.int32),
        pltpu.VMEM((ROW,), jnp.float32),
        pltpu.VMEM_SHARED((ACC_ROWS, ROW), jnp.float32),
        pltpu.SemaphoreType.DMA((NBUF,)),
        pltpu.SemaphoreType.DMA((NBUF,)),
    ],
    compiler_params=pltpu.CompilerParams(
        use_tc_tiling_on_sc=False, needs_layout_passes=False),
)
def _sc_pool(x_hbm, w_hbm, ids_hbm, out_hbm, *scratch):
    _sc_pool_body(x_hbm, w_hbm, ids_hbm, out_hbm, *scratch)


def kernel(x, batch, positions, W1, b1, W2, b2):
    weights = _compute_weights(positions, W1, b1, W2, b2)
    ids = batch.astype(jnp.int32)
    partials = _sc_pool(x.reshape(N, ROW), weights, ids)
    pooled = _combine(partials)
    out = pooled.reshape(NUM_SEGMENTS, D, PROJ)
    return jnp.transpose(out, (0, 2, 1))[..., None]


# speedup vs baseline: 2.0496x; 1.1564x over previous
import functools

import jax
import jax.numpy as jnp
from jax import lax
from jax.experimental import pallas as pl
from jax.experimental.pallas import tpu as pltpu
from jax.experimental.pallas import tpu_sc as plsc

N = 100000
D = 8
PROJ = 64
POS_DIM = 3
NUM_SEGMENTS = 512
ROW = D * PROJ

WBLK = 2000

NCORE = 2
NSUB = 16
NW = NCORE * NSUB
CH = 3136
BB = 32
NB_EFF = 98
NBUF = 2
DUMMY = NUM_SEGMENTS
ACC_ROWS = 544
NVEC = ROW // 16
NEG = -2147483647


def _weights_body(pos_ref, w1_ref, b1_ref, w2_ref, b2_ref, out_ref):
    pos = pos_ref[...]
    h = jnp.dot(pos, w1_ref[...], preferred_element_type=jnp.float32) + b1_ref[...]
    h = jnp.where(h > 0, h, jnp.exp(h) - 1.0)
    w = jnp.dot(h, w2_ref[...], preferred_element_type=jnp.float32) + b2_ref[...]
    w = w - jnp.max(w, axis=-1, keepdims=True)
    w = jnp.exp(w)
    out_ref[...] = w / jnp.sum(w, axis=-1, keepdims=True)


def _compute_weights(positions, W1, b1, W2, b2):
    nblocks = N // WBLK
    return pl.pallas_call(
        _weights_body,
        grid=(nblocks,),
        in_specs=[
            pl.BlockSpec((WBLK, POS_DIM), lambda i: (i, 0)),
            pl.BlockSpec((POS_DIM, PROJ), lambda i: (0, 0)),
            pl.BlockSpec((1, PROJ), lambda i: (0, 0)),
            pl.BlockSpec((PROJ, PROJ), lambda i: (0, 0)),
            pl.BlockSpec((1, PROJ), lambda i: (0, 0)),
        ],
        out_specs=pl.BlockSpec((WBLK, PROJ), lambda i: (i, 0)),
        out_shape=jax.ShapeDtypeStruct((N, PROJ), jnp.float32),
    )(positions, W1, b1.reshape(1, PROJ), W2, b2.reshape(1, PROJ))


def _combine_body(p_ref, out_ref):
    out_ref[...] = p_ref[0:NUM_SEGMENTS, :] + p_ref[NUM_SEGMENTS:2 * NUM_SEGMENTS, :]


def _combine(partials):
    return pl.pallas_call(
        _combine_body,
        out_shape=jax.ShapeDtypeStruct((NUM_SEGMENTS, ROW), jnp.float32),
    )(partials)


def _sc_pool_body(x_hbm, w_hbm, ids_hbm, out_hbm,
                  xbuf, wbuf, idbuf, pbuf, ibuf, ibuf2, tmpbuf, accbuf, accsh,
                  sseg, semx, semw):
    c = lax.axis_index("c")
    s = lax.axis_index("s")
    wid = c * NSUB + s
    r0 = wid * CH
    nrows = jnp.minimum(CH, N - r0)
    nb = nrows // BB
    i0 = jnp.minimum(r0, N - CH)
    d0 = r0 - i0

    zero16 = jnp.zeros((16,), jnp.float32)
    iota16 = lax.iota(jnp.int32, 16)

    def _zero_acc():
        for j in range(NVEC):
            accbuf[0, pl.ds(16 * j, 16)] = zero16
    _zero_acc()

    def _zrow(i, _):
        pltpu.sync_copy(accbuf, accsh.at[pl.ds(s * (ACC_ROWS // NSUB) + i, 1)])
        return 0
    lax.fori_loop(0, ACC_ROWS // NSUB, _zrow, 0)

    pltpu.sync_copy(ids_hbm.at[pl.ds(pl.multiple_of(i0, 8), CH)], idbuf)
    pltpu.sync_copy(
        ids_hbm.at[pl.ds(pl.multiple_of(jnp.maximum(r0 - 16, 0), 8), 16)], pbuf)
    first_w = jnp.min(idbuf[pl.ds(pl.multiple_of(d0, 8), 16)])
    last_w = jnp.max(idbuf[pl.ds(pl.multiple_of(d0 + nrows - 16, 8), 16)])
    prev_id = jnp.max(pbuf[...])
    owned_first = jnp.logical_or(s == 0, prev_id != first_w)
    ibuf[0, pl.ds(0, 16)] = jnp.where(owned_first,
                                      jnp.full((16,), -1, jnp.int32),
                                      jnp.broadcast_to(first_w, (16,)))
    pltpu.sync_copy(ibuf, sseg.at[pl.ds(s, 1)])
    plsc.subcore_barrier()

    def _issue(b, p):
        bc = jnp.minimum(b, nb - 1)
        base = r0 + bc * BB
        pltpu.make_async_copy(x_hbm.at[pl.ds(base, BB), :], xbuf.at[p], semx.at[p]).start()
        pltpu.make_async_copy(w_hbm.at[pl.ds(base, BB), :], wbuf.at[p], semw.at[p]).start()

    def _wait(b, p):
        bc = jnp.minimum(b, nb - 1)
        base = r0 + bc * BB
        pltpu.make_async_copy(x_hbm.at[pl.ds(base, BB), :], xbuf.at[p], semx.at[p]).wait()
        pltpu.make_async_copy(w_hbm.at[pl.ds(base, BB), :], wbuf.at[p], semw.at[p]).wait()

    def _flush(cur_seg):
        dst = jnp.where(cur_seg < 0, jnp.int32(DUMMY), cur_seg)
        dst = jnp.where(
            jnp.logical_and(cur_seg == first_w, jnp.logical_not(owned_first)),
            jnp.int32(NUM_SEGMENTS + 1) + s, dst)
        pltpu.sync_copy(accbuf, accsh.at[pl.ds(dst, 1)])
        _zero_acc()

    def _accum_run16(p, g):
        for q in range(4):
            acc = [zero16] * D
            for i in range(16):
                r = 16 * g + i
                wq = wbuf[p, r, pl.ds(16 * q, 16)]
                for dd in range(D):
                    acc[dd] = acc[dd] + xbuf[p, r, pl.ds(16 * (4 * dd + q), 16)] * wq
            for dd in range(D):
                plsc.addupdate(accbuf.at[0, pl.ds(16 * (4 * dd + q), 16)], acc[dd])

    def _accum_row(p, r):
        wv = [wbuf[p, r, pl.ds(16 * q, 16)] for q in range(4)]
        for j in range(NVEC):
            plsc.addupdate(accbuf.at[0, pl.ds(16 * j, 16)],
                           xbuf[p, r, pl.ds(16 * j, 16)] * wv[j % 4])

    def _group(cur_seg, p, b, g):
        pos = pl.multiple_of(d0 + jnp.minimum(b * BB + 16 * g, nrows - 16), 8)
        gids = idbuf[pl.ds(pos, 16)]
        gids = jnp.where(b < nb, gids, jnp.full((16,), DUMMY, jnp.int32))
        first = jnp.min(gids)
        last = jnp.max(gids)

        def _uni(cur):
            @pl.when(first != cur)
            def _():
                _flush(cur)
            _accum_run16(p, g)
            return first

        def _slow(cur):
            def _row(i, cur):
                sid = jnp.max(jnp.where(iota16 == i, gids, jnp.int32(NEG)))

                @pl.when(sid != cur)
                def _():
                    _flush(cur)
                _accum_row(p, 16 * g + i)
                return sid
            return lax.fori_loop(0, 16, _row, cur)

        return lax.cond(first == last, _uni, _slow, cur_seg)

    for p in range(NBUF):
        _issue(jnp.int32(p), p)

    def _super(i, cur_seg):
        for p in range(NBUF):
            b = NBUF * i + p
            _wait(b, p)
            cur_seg = lax.fori_loop(0, BB // 16,
                                    lambda g, cc: _group(cc, p, b, g), cur_seg)

            @pl.when(b + NBUF < NB_EFF)
            def _():
                _issue(b + NBUF, p)
        return cur_seg

    cur_seg = lax.fori_loop(0, NB_EFF // NBUF, _super, jnp.int32(-1))
    _flush(cur_seg)

    plsc.subcore_barrier()

    own_last = jnp.logical_or(owned_first, last_w != first_w)
    pltpu.sync_copy(sseg, ibuf2)
    _zero_acc()
    cnt = jnp.int32(0)
    for t in range(1, NSUB):
        stag = jnp.max(ibuf2[t, pl.ds(0, 16)])
        m = jnp.logical_and(jnp.logical_and(t > s, own_last), stag == last_w)

        @pl.when(m)
        def _():
            pltpu.sync_copy(accsh.at[pl.ds(NUM_SEGMENTS + 1 + t, 1)], tmpbuf)
            for j in range(NVEC):
                plsc.addupdate(accbuf.at[0, pl.ds(16 * j, 16)],
                               tmpbuf[0, pl.ds(16 * j, 16)])
        cnt = cnt + jnp.where(m, 1, 0)

    @pl.when(cnt > 0)
    def _pickup():
        pltpu.sync_copy(accsh.at[pl.ds(last_w, 1)], tmpbuf)
        for j in range(NVEC):
            plsc.addupdate(accbuf.at[0, pl.ds(16 * j, 16)],
                           tmpbuf[0, pl.ds(16 * j, 16)])
        pltpu.sync_copy(accbuf, accsh.at[pl.ds(last_w, 1)])

    plsc.subcore_barrier()

    o0 = c * NUM_SEGMENTS + s * 32
    pltpu.sync_copy(accsh.at[pl.ds(s * 32, 32)], out_hbm.at[pl.ds(o0, 32), :])


@functools.partial(
    pl.kernel,
    out_type=jax.ShapeDtypeStruct((NCORE * NUM_SEGMENTS, ROW), jnp.float32),
    mesh=plsc.VectorSubcoreMesh(core_axis_name="c", subcore_axis_name="s"),
    scratch_types=[
        pltpu.VMEM((NBUF, BB, ROW), jnp.float32),
        pltpu.VMEM((NBUF, BB, PROJ), jnp.float32),
        pltpu.VMEM((CH,), jnp.int32),
        pltpu.VMEM((16,), jnp.int32),
        pltpu.VMEM((1, 16), jnp.int32),
        pltpu.VMEM((NSUB, 16), jnp.int32),
        pltpu.VMEM((1, ROW), jnp.float32),
        pltpu.VMEM((1, ROW), jnp.float32),
        pltpu.VMEM_SHARED((ACC_ROWS, ROW), jnp.float32),
        pltpu.VMEM_SHARED((NSUB, 16), jnp.int32),
        pltpu.SemaphoreType.DMA((NBUF,)),
        pltpu.SemaphoreType.DMA((NBUF,)),
    ],
    compiler_params=pltpu.CompilerParams(
        use_tc_tiling_on_sc=True, needs_layout_passes=False),
)
def _sc_pool(x_hbm, w_hbm, ids_hbm, out_hbm, *scratch):
    _sc_pool_body(x_hbm, w_hbm, ids_hbm, out_hbm, *scratch)


def kernel(x, batch, positions, W1, b1, W2, b2):
    weights = _compute_weights(positions, W1, b1, W2, b2)
    ids = batch.astype(jnp.int32)
    partials = _sc_pool(x.reshape(N, ROW), weights, ids)
    pooled = _combine(partials)
    out = pooled.reshape(NUM_SEGMENTS, D, PROJ)
    return jnp.transpose(out, (0, 2, 1))[..., None]


# speedup vs baseline: 2.0509x; 1.0006x over previous
import functools

import jax
import jax.numpy as jnp
from jax import lax
from jax.experimental import pallas as pl
from jax.experimental.pallas import tpu as pltpu
from jax.experimental.pallas import tpu_sc as plsc

N = 100000
D = 8
PROJ = 64
POS_DIM = 3
NUM_SEGMENTS = 512
ROW = D * PROJ

WBLK = 2000

NCORE = 2
NSUB = 16
NW = NCORE * NSUB
CH = 3136
BB = 32
NB_EFF = 98
NBUF = 2
DUMMY = NUM_SEGMENTS
ACC_ROWS = 544
NVEC = ROW // 16
NEG = -2147483647


def _weights_body(pos_ref, w1_ref, b1_ref, w2_ref, b2_ref, out_ref):
    pos = pos_ref[...]
    h = jnp.dot(pos, w1_ref[...], preferred_element_type=jnp.float32) + b1_ref[...]
    h = jnp.where(h > 0, h, jnp.exp(h) - 1.0)
    w = jnp.dot(h, w2_ref[...], preferred_element_type=jnp.float32) + b2_ref[...]
    w = w - jnp.max(w, axis=-1, keepdims=True)
    w = jnp.exp(w)
    out_ref[...] = w / jnp.sum(w, axis=-1, keepdims=True)


def _compute_weights(positions, W1, b1, W2, b2):
    nblocks = N // WBLK
    return pl.pallas_call(
        _weights_body,
        grid=(nblocks,),
        in_specs=[
            pl.BlockSpec((WBLK, POS_DIM), lambda i: (i, 0)),
            pl.BlockSpec((POS_DIM, PROJ), lambda i: (0, 0)),
            pl.BlockSpec((1, PROJ), lambda i: (0, 0)),
            pl.BlockSpec((PROJ, PROJ), lambda i: (0, 0)),
            pl.BlockSpec((1, PROJ), lambda i: (0, 0)),
        ],
        out_specs=pl.BlockSpec((WBLK, PROJ), lambda i: (i, 0)),
        out_shape=jax.ShapeDtypeStruct((N, PROJ), jnp.float32),
    )(positions, W1, b1.reshape(1, PROJ), W2, b2.reshape(1, PROJ))


def _combine_body(p_ref, out_ref):
    out_ref[...] = p_ref[0:NUM_SEGMENTS, :] + p_ref[NUM_SEGMENTS:2 * NUM_SEGMENTS, :]


def _combine(partials):
    return pl.pallas_call(
        _combine_body,
        out_shape=jax.ShapeDtypeStruct((NUM_SEGMENTS, ROW), jnp.float32),
    )(partials)


def _sc_pool_body(x_hbm, w_hbm, ids_hbm, out_hbm,
                  xbuf, wbuf, idbuf, pbuf, ibuf, ibuf2, tmpbuf, accbuf, accsh,
                  sseg, semx, semw):
    c = lax.axis_index("c")
    s = lax.axis_index("s")
    wid = c * NSUB + s
    r0 = wid * CH
    nrows = jnp.minimum(CH, N - r0)
    nb = nrows // BB
    i0 = jnp.minimum(r0, N - CH)
    d0 = r0 - i0

    zero16 = jnp.zeros((16,), jnp.float32)
    iota16 = lax.iota(jnp.int32, 16)

    def _zero_acc():
        for j in range(NVEC):
            accbuf[0, pl.ds(16 * j, 16)] = zero16
    _zero_acc()

    def _zrow(i, _):
        pltpu.sync_copy(accbuf, accsh.at[pl.ds(s * (ACC_ROWS // NSUB) + i, 1)])
        return 0
    lax.fori_loop(0, ACC_ROWS // NSUB, _zrow, 0)

    pltpu.sync_copy(ids_hbm.at[pl.ds(pl.multiple_of(i0, 8), CH)], idbuf)
    pltpu.sync_copy(
        ids_hbm.at[pl.ds(pl.multiple_of(jnp.maximum(r0 - 16, 0), 8), 16)], pbuf)
    first_w = jnp.min(idbuf[pl.ds(pl.multiple_of(d0, 8), 16)])
    last_w = jnp.max(idbuf[pl.ds(pl.multiple_of(d0 + nrows - 16, 8), 16)])
    prev_id = jnp.max(pbuf[...])
    owned_first = jnp.logical_or(s == 0, prev_id != first_w)
    ibuf[pl.ds(0, 16)] = jnp.where(owned_first,
                                   jnp.full((16,), -1, jnp.int32),
                                   jnp.broadcast_to(first_w, (16,)))
    pltpu.sync_copy(ibuf, sseg.at[pl.ds(pl.multiple_of(s * 16, 8), 16)])
    plsc.subcore_barrier()

    def _issue(b, p):
        bc = jnp.minimum(b, nb - 1)
        base = r0 + bc * BB
        pltpu.make_async_copy(x_hbm.at[pl.ds(base, BB), :], xbuf.at[p], semx.at[p]).start()
        pltpu.make_async_copy(w_hbm.at[pl.ds(base, BB), :], wbuf.at[p], semw.at[p]).start()

    def _wait(b, p):
        bc = jnp.minimum(b, nb - 1)
        base = r0 + bc * BB
        pltpu.make_async_copy(x_hbm.at[pl.ds(base, BB), :], xbuf.at[p], semx.at[p]).wait()
        pltpu.make_async_copy(w_hbm.at[pl.ds(base, BB), :], wbuf.at[p], semw.at[p]).wait()

    def _flush(cur_seg):
        dst = jnp.where(cur_seg < 0, jnp.int32(DUMMY), cur_seg)
        dst = jnp.where(
            jnp.logical_and(cur_seg == first_w, jnp.logical_not(owned_first)),
            jnp.int32(NUM_SEGMENTS + 1) + s, dst)
        pltpu.sync_copy(accbuf, accsh.at[pl.ds(dst, 1)])
        _zero_acc()

    def _accum_run16(p, g):
        for q in range(4):
            acc = [zero16] * D
            for i in range(16):
                r = 16 * g + i
                wq = wbuf[p, r, pl.ds(16 * q, 16)]
                for dd in range(D):
                    acc[dd] = acc[dd] + xbuf[p, r, pl.ds(16 * (4 * dd + q), 16)] * wq
            for dd in range(D):
                plsc.addupdate(accbuf.at[0, pl.ds(16 * (4 * dd + q), 16)], acc[dd])

    def _accum_row(p, r):
        wv = [wbuf[p, r, pl.ds(16 * q, 16)] for q in range(4)]
        for j in range(NVEC):
            plsc.addupdate(accbuf.at[0, pl.ds(16 * j, 16)],
                           xbuf[p, r, pl.ds(16 * j, 16)] * wv[j % 4])

    def _group(cur_seg, p, b, g):
        pos = pl.multiple_of(d0 + jnp.minimum(b * BB + 16 * g, nrows - 16), 8)
        gids = idbuf[pl.ds(pos, 16)]
        gids = jnp.where(b < nb, gids, jnp.full((16,), DUMMY, jnp.int32))
        first = jnp.min(gids)
        last = jnp.max(gids)

        def _uni(cur):
            @pl.when(first != cur)
            def _():
                _flush(cur)
            _accum_run16(p, g)
            return first

        def _slow(cur):
            def _row(i, cur):
                sid = jnp.max(jnp.where(iota16 == i, gids, jnp.int32(NEG)))

                @pl.when(sid != cur)
                def _():
                    _flush(cur)
                _accum_row(p, 16 * g + i)
                return sid
            return lax.fori_loop(0, 16, _row, cur)

        return lax.cond(first == last, _uni, _slow, cur_seg)

    for p in range(NBUF):
        _issue(jnp.int32(p), p)

    def _super(i, cur_seg):
        for p in range(NBUF):
            b = NBUF * i + p
            _wait(b, p)
            cur_seg = lax.fori_loop(0, BB // 16,
                                    lambda g, cc: _group(cc, p, b, g), cur_seg)

            @pl.when(b + NBUF < NB_EFF)
            def _():
                _issue(b + NBUF, p)
        return cur_seg

    cur_seg = lax.fori_loop(0, NB_EFF // NBUF, _super, jnp.int32(-1))
    _flush(cur_seg)

    plsc.subcore_barrier()

    own_last = jnp.logical_or(owned_first, last_w != first_w)
    pltpu.sync_copy(sseg, ibuf2)
    _zero_acc()
    cnt = jnp.int32(0)
    for t in range(1, NSUB):
        stag = jnp.max(ibuf2[pl.ds(16 * t, 16)])
        m = jnp.logical_and(jnp.logical_and(t > s, own_last), stag == last_w)

        @pl.when(m)
        def _():
            pltpu.sync_copy(accsh.at[pl.ds(NUM_SEGMENTS + 1 + t, 1)], tmpbuf)
            for j in range(NVEC):
                plsc.addupdate(accbuf.at[0, pl.ds(16 * j, 16)],
                               tmpbuf[0, pl.ds(16 * j, 16)])
        cnt = cnt + jnp.where(m, 1, 0)

    @pl.when(cnt > 0)
    def _pickup():
        pltpu.sync_copy(accsh.at[pl.ds(last_w, 1)], tmpbuf)
        for j in range(NVEC):
            plsc.addupdate(accbuf.at[0, pl.ds(16 * j, 16)],
                           tmpbuf[0, pl.ds(16 * j, 16)])
        pltpu.sync_copy(accbuf, accsh.at[pl.ds(last_w, 1)])

    plsc.subcore_barrier()

    o0 = c * NUM_SEGMENTS + s * 32
    pltpu.sync_copy(accsh.at[pl.ds(s * 32, 32)], out_hbm.at[pl.ds(o0, 32), :])


@functools.partial(
    pl.kernel,
    out_type=jax.ShapeDtypeStruct((NCORE * NUM_SEGMENTS, ROW), jnp.float32),
    mesh=plsc.VectorSubcoreMesh(core_axis_name="c", subcore_axis_name="s"),
    scratch_types=[
        pltpu.VMEM((NBUF, BB, ROW), jnp.float32),
        pltpu.VMEM((NBUF, BB, PROJ), jnp.float32),
        pltpu.VMEM((CH,), jnp.int32),
        pltpu.VMEM((16,), jnp.int32),
        pltpu.VMEM((16,), jnp.int32),
        pltpu.VMEM((NSUB * 16,), jnp.int32),
        pltpu.VMEM((1, ROW), jnp.float32),
        pltpu.VMEM((1, ROW), jnp.float32),
        pltpu.VMEM_SHARED((ACC_ROWS, ROW), jnp.float32),
        pltpu.VMEM_SHARED((NSUB * 16,), jnp.int32),
        pltpu.SemaphoreType.DMA((NBUF,)),
        pltpu.SemaphoreType.DMA((NBUF,)),
    ],
    compiler_params=pltpu.CompilerParams(
        use_tc_tiling_on_sc=True, needs_layout_passes=False),
)
def _sc_pool(x_hbm, w_hbm, ids_hbm, out_hbm, *scratch):
    _sc_pool_body(x_hbm, w_hbm, ids_hbm, out_hbm, *scratch)


def kernel(x, batch, positions, W1, b1, W2, b2):
    weights = _compute_weights(positions, W1, b1, W2, b2)
    ids = batch.astype(jnp.int32)
    partials = _sc_pool(x.reshape(N, ROW), weights, ids)
    pooled = _combine(partials)
    out = pooled.reshape(NUM_SEGMENTS, D, PROJ)
    return jnp.transpose(out, (0, 2, 1))[..., None]
